# baseline (speedup 1.0000x reference)
import jax
import jax.numpy as jnp
from jax import lax
from jax.experimental import pallas as pl
from jax.experimental.pallas import tpu as pltpu
from jax.experimental.pallas import tpu_sc as plsc

E = 100000
R = 500
S_DIM = 96
T_DIM = 32
B = 4096

NC = 2
NS = 16
NW = NC * NS
BPW = B // NW

PACK_BR = 2000
PACK_W = 256
TC_BLK = 512


def _repack_a_body(ent, yf, mf, df, yp, mp, out):
    out[:, 0:96] = ent[...]
    out[:, 96:128] = yf[...]
    out[:, 128:160] = mf[...]
    out[:, 160:192] = df[...]
    out[:, 192:224] = yp[...]
    out[:, 224:256] = mp[...]


def _repack_b_body(etr, dp, ya, ma, da, out):
    out[:, 0:96] = etr[...]
    out[:, 96:128] = dp[...]
    out[:, 128:160] = ya[...]
    out[:, 160:192] = ma[...]
    out[:, 192:224] = da[...]
    out[:, 224:256] = da[...]


def _repack(body, wide, *narrow):
    widths = [wide.shape[1]] + [n.shape[1] for n in narrow]
    return pl.pallas_call(
        body,
        grid=(E // PACK_BR,),
        in_specs=[pl.BlockSpec((PACK_BR, w), lambda i: (i, 0)) for w in widths],
        out_specs=pl.BlockSpec((PACK_BR, PACK_W), lambda i: (i, 0)),
        out_shape=jax.ShapeDtypeStruct((E, PACK_W), jnp.float32),
    )(wide, *narrow)


def _mesh():
    return plsc.VectorSubcoreMesh(core_axis_name="c", subcore_axis_name="s")


def _gather2_body(tab_a, tab_b, idx_a_hbm, idx_b_hbm, out_a, out_b,
                  idx_a, idx_b, buf0, buf1, sem0, sem1):
    cid = lax.axis_index("c")
    sid = lax.axis_index("s")
    base = (sid * NC + cid) * BPW

    pltpu.sync_copy(idx_a_hbm.at[pl.ds(base, BPW)], idx_a)
    pltpu.sync_copy(idx_b_hbm.at[pl.ds(base, BPW)], idx_b)

    c0 = pltpu.make_async_copy(tab_a.at[idx_a], buf0, sem0)
    c0.start()
    c1 = pltpu.make_async_copy(tab_b.at[idx_b], buf1, sem1)
    c1.start()
    c0.wait()
    pltpu.sync_copy(buf0, out_a.at[pl.ds(base, BPW)])
    c1.wait()
    pltpu.sync_copy(buf1, out_b.at[pl.ds(base, BPW)])


def _gather2(tab_a, tab_b, idx_a, idx_b):
    w_a = tab_a.shape[1]
    w_b = tab_b.shape[1]
    kern = pl.kernel(
        _gather2_body,
        out_type=[jax.ShapeDtypeStruct((B, w_a), jnp.float32),
                  jax.ShapeDtypeStruct((B, w_b), jnp.float32)],
        mesh=_mesh(),
        scratch_types=[
            pltpu.VMEM((BPW,), jnp.int32),
            pltpu.VMEM((BPW,), jnp.int32),
            pltpu.VMEM((BPW, w_a), jnp.float32),
            pltpu.VMEM((BPW, w_b), jnp.float32),
            pltpu.SemaphoreType.DMA,
            pltpu.SemaphoreType.DMA,
        ],
    )
    return kern(tab_a, tab_b, idx_a, idx_b)


def _qmul(a, b):
    sa, xa, ya, za = a
    sb, xb, yb, zb = b
    return (sa * sb - xa * xb - ya * yb - za * zb,
            sa * xb + sb * xa + ya * zb - yb * za,
            sa * yb + sb * ya + za * xb - zb * xa,
            sa * zb + sb * za + xa * yb - xb * ya)


def _qnorm(b):
    sb, xb, yb, zb = b
    inv = lax.rsqrt(sb * sb + xb * xb + yb * yb + zb * zb)
    return (sb * inv, xb * inv, yb * inv, zb * inv)


def _tc_body(y_r, m_r, d_r, gah_r, gat_r, gbh_r, gbt_r, r_r, rtr_r, out):
    y = y_r[...]
    m = m_r[...]
    d = d_r[...]

    def time_emb(ga, gb):
        yf = ga[:, 96:128]
        mf = ga[:, 128:160]
        df = ga[:, 160:192]
        yp = ga[:, 192:224]
        mp = ga[:, 224:256]
        dp = gb[:, 96:128]
        ya = gb[:, 128:160]
        ma = gb[:, 160:192]
        da = gb[:, 192:224]
        return (ya * jnp.sin(yf * y + yp)
                + ma * jnp.sin(mf * m + mp)
                + da * jnp.sin(df * d + dp))

    gah = gah_r[...]
    gat = gat_r[...]
    gbh = gbh_r[...]
    gbt = gbt_r[...]

    th = time_emb(gah, gbh)
    tt = time_emb(gat, gbt)

    h = (gah[:, 0:32], gah[:, 32:64], gah[:, 64:96], th)
    h_tr = (gbh[:, 0:32], gbh[:, 32:64], gbh[:, 64:96], th)
    t = (gat[:, 0:32], gat[:, 32:64], gat[:, 64:96], tt)
    t_tr = (gbt[:, 0:32], gbt[:, 32:64], gbt[:, 64:96], tt)

    rv = r_r[...]
    rtrv = rtr_r[...]
    rq = (rv[:, 0:32], rv[:, 32:64], rv[:, 64:96], rv[:, 96:128])
    rtrq = (rtrv[:, 0:32], rtrv[:, 32:64], rtrv[:, 64:96], rtrv[:, 96:128])
    nrtr = _qnorm(rtrq)
    nr = _qnorm(rq)

    h1 = _qmul(_qmul(h, _qnorm(h_tr)), nrtr)
    t1 = _qmul(_qmul(t, _qnorm(t_tr)), nrtr)
    hr = _qmul(h1, nr)

    acc = (hr[0] * t1[0] + hr[1] * t1[1] + hr[2] * t1[2] + hr[3] * t1[3])
    out[...] = jnp.sum(acc, axis=1, keepdims=True)


def _tc_compute(years, months, days, gah, gat, gbh, gbt, r, rtr):
    widths = [1, 1, 1, PACK_W, PACK_W, PACK_W, PACK_W, 128, 128]
    return pl.pallas_call(
        _tc_body,
        grid=(B // TC_BLK,),
        in_specs=[pl.BlockSpec((TC_BLK, w), lambda i: (i, 0)) for w in widths],
        out_specs=pl.BlockSpec((TC_BLK, 1), lambda i: (i, 0)),
        out_shape=jax.ShapeDtypeStruct((B, 1), jnp.float32),
    )(years.reshape(B, 1), months.reshape(B, 1), days.reshape(B, 1),
      gah, gat, gbh, gbt, r, rtr)


def kernel(heads, rels, tails, years, months, days, ent_embs, rel_embs,
           ent_transfer, rel_transfer, y_freq, m_freq, d_freq, y_phi, m_phi,
           d_phi, y_amp, m_amp, d_amp):
    heads = heads.astype(jnp.int32)
    tails = tails.astype(jnp.int32)
    rels = rels.astype(jnp.int32)

    r, rtr = _gather2(rel_embs, rel_transfer, rels, rels)
    pa = _repack(_repack_a_body, ent_embs, y_freq, m_freq, d_freq, y_phi,
                 m_phi)
    gah, gat = _gather2(pa, pa, heads, tails)
    pb = _repack(_repack_b_body, ent_transfer, d_phi, y_amp, m_amp, d_amp)
    gbh, gbt = _gather2(pb, pb, heads, tails)

    score = _tc_compute(years, months, days, gah, gat, gbh, gbt, r, rtr)
    return score.reshape(B)
